# baseline (speedup 1.0000x reference)
import functools

import jax
import jax.numpy as jnp
from jax import lax
from jax.experimental import pallas as pl
from jax.experimental.pallas import tpu as pltpu

_KT = 1024


def _layer0_kernel(x_ref, w0_ref, h_ref, *, chunk):
    k = pl.program_id(1)
    xb = x_ref[...].astype(jnp.bfloat16)
    nblk = h_ref.shape[1]

    for c in range(0, nblk, chunk):
        w0b = w0_ref[pl.ds(c, chunk), :].astype(jnp.bfloat16)
        part = lax.dot_general(xb, w0b, (((1,), (1,)), ((), ())),
                               preferred_element_type=jnp.float32)

        @pl.when(k == 0)
        def _init():
            h_ref[:, pl.ds(c, chunk)] = part

        @pl.when(k != 0)
        def _accum():
            h_ref[:, pl.ds(c, chunk)] += part


def _tail_kernel(h_ref, w1_ref, w2_ref, b0_ref, b1_ref, b2_ref, out_ref):
    h1 = jnp.maximum(h_ref[...] + b0_ref[...], 0.0).astype(jnp.bfloat16)
    h2 = lax.dot_general(h1, w1_ref[...], (((1,), (1,)), ((), ())),
                         preferred_element_type=jnp.float32)
    h2 = jnp.maximum(h2 + b1_ref[...], 0.0).astype(jnp.bfloat16)
    o = lax.dot_general(h2, w2_ref[...], (((1,), (1,)), ((), ())),
                        preferred_element_type=jnp.float32)
    out_ref[...] = o + b2_ref[...]


def kernel(x, W0, b0, W1, b1, W2, b2):
    B, F0 = x.shape
    F1 = W0.shape[0]
    F2 = W1.shape[0]
    F3 = W2.shape[0]
    kt = min(_KT, F0)
    nk = F0 // kt
    nblk = min(2048, F1)
    nn = F1 // nblk
    chunk = min(1024, nblk)

    h1 = pl.pallas_call(
        functools.partial(_layer0_kernel, chunk=chunk),
        grid=(nn, nk),
        in_specs=[
            pl.BlockSpec((B, kt), lambda n, k: (0, k)),
            pl.BlockSpec((nblk, kt), lambda n, k: (n, k)),
        ],
        out_specs=pl.BlockSpec((B, nblk), lambda n, k: (0, n)),
        out_shape=jax.ShapeDtypeStruct((B, F1), jnp.float32),
        compiler_params=pltpu.CompilerParams(
            dimension_semantics=("parallel", "arbitrary"),
        ),
    )(x, W0)

    w1b = W1.astype(jnp.bfloat16)
    w2b = W2.astype(jnp.bfloat16)
    b0r = b0.reshape(1, F1)
    b1r = b1.reshape(1, F2)
    b2r = b2.reshape(1, F3)

    rows = 128
    return pl.pallas_call(
        _tail_kernel,
        grid=(B // rows,),
        in_specs=[
            pl.BlockSpec((rows, F1), lambda i: (i, 0)),
            pl.BlockSpec((F2, F1), lambda i: (0, 0)),
            pl.BlockSpec((F3, F2), lambda i: (0, 0)),
            pl.BlockSpec((1, F1), lambda i: (0, 0)),
            pl.BlockSpec((1, F2), lambda i: (0, 0)),
            pl.BlockSpec((1, F3), lambda i: (0, 0)),
        ],
        out_specs=pl.BlockSpec((rows, F3), lambda i: (i, 0)),
        out_shape=jax.ShapeDtypeStruct((B, F3), jnp.float32),
        compiler_params=pltpu.CompilerParams(
            dimension_semantics=("parallel",),
        ),
    )(h1, w1b, w2b, b0r, b1r, b2r)


# speedup vs baseline: 1.1468x; 1.1468x over previous
import functools

import jax
import jax.numpy as jnp
from jax import lax
from jax.experimental import pallas as pl
from jax.experimental.pallas import tpu as pltpu

_KT = 1024


def _layer0_kernel(x_ref, w0_ref, h_ref, *, chunk):
    k = pl.program_id(1)
    xb = x_ref[...].astype(jnp.bfloat16)
    nblk = h_ref.shape[1]

    for c in range(0, nblk, chunk):
        w0b = w0_ref[pl.ds(c, chunk), :].astype(jnp.bfloat16)
        part = lax.dot_general(xb, w0b, (((1,), (1,)), ((), ())),
                               preferred_element_type=jnp.float32)
        cur = h_ref[:, pl.ds(c, chunk)]
        h_ref[:, pl.ds(c, chunk)] = jnp.where(k == 0, part, cur + part)


def _tail_kernel(h_ref, w1_ref, w2_ref, b0_ref, b1_ref, b2_ref, out_ref):
    h1 = jnp.maximum(h_ref[...] + b0_ref[...], 0.0).astype(jnp.bfloat16)
    h2 = lax.dot_general(h1, w1_ref[...], (((1,), (1,)), ((), ())),
                         preferred_element_type=jnp.float32)
    h2 = jnp.maximum(h2 + b1_ref[...], 0.0).astype(jnp.bfloat16)
    o = lax.dot_general(h2, w2_ref[...], (((1,), (1,)), ((), ())),
                        preferred_element_type=jnp.float32)
    out_ref[...] = o + b2_ref[...]


def kernel(x, W0, b0, W1, b1, W2, b2):
    B, F0 = x.shape
    F1 = W0.shape[0]
    F2 = W1.shape[0]
    F3 = W2.shape[0]
    kt = min(_KT, F0)
    nk = F0 // kt
    nblk = min(2048, F1)
    nn = F1 // nblk
    chunk = min(1024, nblk)

    h1 = pl.pallas_call(
        functools.partial(_layer0_kernel, chunk=chunk),
        grid=(nn, nk),
        in_specs=[
            pl.BlockSpec((B, kt), lambda n, k: (0, k)),
            pl.BlockSpec((nblk, kt), lambda n, k: (n, k)),
        ],
        out_specs=pl.BlockSpec((B, nblk), lambda n, k: (0, n)),
        out_shape=jax.ShapeDtypeStruct((B, F1), jnp.float32),
        compiler_params=pltpu.CompilerParams(
            dimension_semantics=("parallel", "arbitrary"),
        ),
    )(x, W0)

    w1b = W1.astype(jnp.bfloat16)
    w2b = W2.astype(jnp.bfloat16)
    b0r = b0.reshape(1, F1)
    b1r = b1.reshape(1, F2)
    b2r = b2.reshape(1, F3)

    rows = 128
    return pl.pallas_call(
        _tail_kernel,
        grid=(B // rows,),
        in_specs=[
            pl.BlockSpec((rows, F1), lambda i: (i, 0)),
            pl.BlockSpec((F2, F1), lambda i: (0, 0)),
            pl.BlockSpec((F3, F2), lambda i: (0, 0)),
            pl.BlockSpec((1, F1), lambda i: (0, 0)),
            pl.BlockSpec((1, F2), lambda i: (0, 0)),
            pl.BlockSpec((1, F3), lambda i: (0, 0)),
        ],
        out_specs=pl.BlockSpec((rows, F3), lambda i: (i, 0)),
        out_shape=jax.ShapeDtypeStruct((B, F3), jnp.float32),
        compiler_params=pltpu.CompilerParams(
            dimension_semantics=("parallel",),
        ),
    )(h1, w1b, w2b, b0r, b1r, b2r)
